# baseline (speedup 1.0000x reference)
import functools

import jax
import jax.numpy as jnp
from jax import lax
from jax.experimental import pallas as pl
from jax.experimental.pallas import tpu as pltpu
from jax.experimental.pallas import tpu_sc as plsc

N_ROWS = 4096
N_COLS = 4096
NC = 2
NS = 16
NW = NC * NS
ROWS_PER_W = N_ROWS // NW
R = 8
NCHUNK = ROWS_PER_W // R
LANES = 16
HCOLS = N_COLS // 2
HG = HCOLS // LANES
U = 4


@functools.partial(
    pl.kernel,
    mesh=plsc.VectorSubcoreMesh(core_axis_name="c", subcore_axis_name="s"),
    out_type=jax.ShapeDtypeStruct((N_ROWS, N_COLS), jnp.float32),
    compiler_params=pltpu.CompilerParams(needs_layout_passes=False),
    scratch_types=[
        pltpu.VMEM((N_COLS,), jnp.int32),
        pltpu.VMEM((R, N_COLS), jnp.float32),
        pltpu.VMEM((R, N_COLS), jnp.float32),
        pltpu.VMEM((R, HCOLS), jnp.float32),
        pltpu.VMEM((R, HCOLS), jnp.float32),
        pltpu.SemaphoreType.DMA,
        pltpu.SemaphoreType.DMA,
        pltpu.SemaphoreType.DMA,
        pltpu.SemaphoreType.DMA,
    ],
)
def _permute(x_hbm, perm_hbm, out_hbm, perm_v,
             in0, in1, outh0, outh1, isem0, isem1, osem0, osem1):
    wid = lax.axis_index("s") * NC + lax.axis_index("c")
    row0 = wid * ROWS_PER_W

    in_bufs = (in0, in1)
    in_sems = (isem0, isem1)
    out_bufs = (outh0, outh1)
    out_sems = (osem0, osem1)
    row_ids = [jnp.full((LANES,), r, jnp.int32) for r in range(R)]

    def start_in(c, b):
        pltpu.async_copy(x_hbm.at[pl.ds(row0 + c * R, R)], in_bufs[b],
                         in_sems[b])

    def wait_in(c, b):
        pltpu.make_async_copy(x_hbm.at[pl.ds(row0 + c * R, R)], in_bufs[b],
                              in_sems[b]).wait()

    def out_dst(c, h):
        return out_hbm.at[pl.ds(row0 + c * R, R), pl.ds(h * HCOLS, HCOLS)]

    def start_out(c, h):
        pltpu.async_copy(out_bufs[h], out_dst(c, h), out_sems[h])

    def wait_out(c, h):
        pltpu.make_async_copy(out_bufs[h], out_dst(c, h), out_sems[h]).wait()

    def gather_half(src_v, h):
        dst_v = out_bufs[h]

        @plsc.parallel_loop(0, HG // U)
        def _(gu):
            g0 = gu * U
            idxs = [perm_v[pl.ds((h * HG + g0 + u) * LANES, LANES)]
                    for u in range(U)]
            vals = [[plsc.load_gather(src_v, [row_ids[r], idxs[u]])
                     for r in range(R)] for u in range(U)]
            for u in range(U):
                for r in range(R):
                    dst_v[r, pl.ds((g0 + u) * LANES, LANES)] = vals[u][r]

    start_in(0, 0)
    start_in(1, 1)
    pltpu.sync_copy(perm_hbm, perm_v)
    wait_in(0, 0)
    for h in range(2):
        gather_half(in_bufs[0], h)
        start_out(0, h)
    start_in(2, 0)

    @pl.loop(0, (NCHUNK - 2) // 2)
    def _(cc):
        for b in range(2):
            c = 1 + cc * 2 + b
            bb = (1 + b) % 2
            wait_in(c, bb)
            for h in range(2):
                wait_out(c - 1, h)
                gather_half(in_bufs[bb], h)
                start_out(c, h)

            @pl.when(c + 2 < NCHUNK)
            def _():
                start_in(c + 2, bb)

    c = NCHUNK - 1
    wait_in(c, 1)
    for h in range(2):
        wait_out(c - 1, h)
        gather_half(in_bufs[1], h)
        start_out(c, h)
    for h in range(2):
        wait_out(c, h)


def kernel(x, permutation):
    return _permute(x, permutation)


# speedup vs baseline: 1.0431x; 1.0431x over previous
import functools

import jax
import jax.numpy as jnp
from jax import lax
from jax.experimental import pallas as pl
from jax.experimental.pallas import tpu as pltpu
from jax.experimental.pallas import tpu_sc as plsc

N_ROWS = 4096
N_COLS = 4096
NC = 2
NS = 16
NW = NC * NS
ROWS_PER_W = N_ROWS // NW
R = 8
NCHUNK = ROWS_PER_W // R
LANES = 16
HCOLS = N_COLS // 2
HG = HCOLS // LANES
U = 2


@functools.partial(
    pl.kernel,
    mesh=plsc.VectorSubcoreMesh(core_axis_name="c", subcore_axis_name="s"),
    out_type=jax.ShapeDtypeStruct((N_ROWS, N_COLS), jnp.float32),
    compiler_params=pltpu.CompilerParams(needs_layout_passes=False),
    scratch_types=[
        pltpu.VMEM((N_COLS,), jnp.int32),
        pltpu.VMEM((R, N_COLS), jnp.float32),
        pltpu.VMEM((R, N_COLS), jnp.float32),
        pltpu.VMEM((R, HCOLS), jnp.float32),
        pltpu.VMEM((R, HCOLS), jnp.float32),
        pltpu.SemaphoreType.DMA,
        pltpu.SemaphoreType.DMA,
        pltpu.SemaphoreType.DMA,
        pltpu.SemaphoreType.DMA,
    ],
)
def _permute(x_hbm, perm_hbm, out_hbm, perm_v,
             in0, in1, outh0, outh1, isem0, isem1, osem0, osem1):
    wid = lax.axis_index("s") * NC + lax.axis_index("c")
    row0 = wid * ROWS_PER_W

    in_bufs = (in0, in1)
    in_sems = (isem0, isem1)
    out_bufs = (outh0, outh1)
    out_sems = (osem0, osem1)
    row_ids = [jnp.full((LANES,), r, jnp.int32) for r in range(R)]

    def start_in(c, b):
        pltpu.async_copy(x_hbm.at[pl.ds(row0 + c * R, R)], in_bufs[b],
                         in_sems[b])

    def wait_in(c, b):
        pltpu.make_async_copy(x_hbm.at[pl.ds(row0 + c * R, R)], in_bufs[b],
                              in_sems[b]).wait()

    def out_dst(c, h):
        return out_hbm.at[pl.ds(row0 + c * R, R), pl.ds(h * HCOLS, HCOLS)]

    def start_out(c, h):
        pltpu.async_copy(out_bufs[h], out_dst(c, h), out_sems[h])

    def wait_out(c, h):
        pltpu.make_async_copy(out_bufs[h], out_dst(c, h), out_sems[h]).wait()

    def gather_half(src_v, h):
        dst_v = out_bufs[h]

        @plsc.parallel_loop(0, HG // U)
        def _(gu):
            g0 = gu * U
            idxs = [perm_v[pl.ds((h * HG + g0 + u) * LANES, LANES)]
                    for u in range(U)]
            vals = [[plsc.load_gather(src_v, [row_ids[r], idxs[u]])
                     for r in range(R)] for u in range(U)]
            for u in range(U):
                for r in range(R):
                    dst_v[r, pl.ds((g0 + u) * LANES, LANES)] = vals[u][r]

    start_in(0, 0)
    start_in(1, 1)
    pltpu.sync_copy(perm_hbm, perm_v)
    wait_in(0, 0)
    for h in range(2):
        gather_half(in_bufs[0], h)
        start_out(0, h)
    start_in(2, 0)

    @pl.loop(0, (NCHUNK - 2) // 2)
    def _(cc):
        for b in range(2):
            c = 1 + cc * 2 + b
            bb = (1 + b) % 2
            wait_in(c, bb)
            for h in range(2):
                wait_out(c - 1, h)
                gather_half(in_bufs[bb], h)
                start_out(c, h)

            @pl.when(c + 2 < NCHUNK)
            def _():
                start_in(c + 2, bb)

    c = NCHUNK - 1
    wait_in(c, 1)
    for h in range(2):
        wait_out(c - 1, h)
        gather_half(in_bufs[1], h)
        start_out(c, h)
    for h in range(2):
        wait_out(c, h)


def kernel(x, permutation):
    return _permute(x, permutation)


# speedup vs baseline: 1.0489x; 1.0056x over previous
import functools

import jax
import jax.numpy as jnp
from jax import lax
from jax.experimental import pallas as pl
from jax.experimental.pallas import tpu as pltpu
from jax.experimental.pallas import tpu_sc as plsc

N_ROWS = 4096
N_COLS = 4096
NC = 2
NS = 16
NW = NC * NS
ROWS_PER_W = N_ROWS // NW
R = 8
NCHUNK = ROWS_PER_W // R
LANES = 16
HCOLS = N_COLS // 2
HG = HCOLS // LANES
U = 2


@functools.partial(
    pl.kernel,
    mesh=plsc.VectorSubcoreMesh(core_axis_name="c", subcore_axis_name="s"),
    out_type=jax.ShapeDtypeStruct((N_ROWS, N_COLS), jnp.float32),
    compiler_params=pltpu.CompilerParams(
        needs_layout_passes=False,
        disable_bounds_checks=True,
        disable_semaphore_checks=True,
    ),
    scratch_types=[
        pltpu.VMEM((N_COLS,), jnp.int32),
        pltpu.VMEM((R, N_COLS), jnp.float32),
        pltpu.VMEM((R, N_COLS), jnp.float32),
        pltpu.VMEM((R, HCOLS), jnp.float32),
        pltpu.VMEM((R, HCOLS), jnp.float32),
        pltpu.SemaphoreType.DMA,
        pltpu.SemaphoreType.DMA,
        pltpu.SemaphoreType.DMA,
        pltpu.SemaphoreType.DMA,
    ],
)
def _permute(x_hbm, perm_hbm, out_hbm, perm_v,
             in0, in1, outh0, outh1, isem0, isem1, osem0, osem1):
    wid = lax.axis_index("s") * NC + lax.axis_index("c")
    row0 = wid * ROWS_PER_W

    in_bufs = (in0, in1)
    in_sems = (isem0, isem1)
    out_bufs = (outh0, outh1)
    out_sems = (osem0, osem1)
    row_ids = [jnp.full((LANES,), r, jnp.int32) for r in range(R)]

    def start_in(c, b):
        pltpu.async_copy(x_hbm.at[pl.ds(row0 + c * R, R)], in_bufs[b],
                         in_sems[b])

    def wait_in(c, b):
        pltpu.make_async_copy(x_hbm.at[pl.ds(row0 + c * R, R)], in_bufs[b],
                              in_sems[b]).wait()

    def out_dst(c, h):
        return out_hbm.at[pl.ds(row0 + c * R, R), pl.ds(h * HCOLS, HCOLS)]

    def start_out(c, h):
        pltpu.async_copy(out_bufs[h], out_dst(c, h), out_sems[h])

    def wait_out(c, h):
        pltpu.make_async_copy(out_bufs[h], out_dst(c, h), out_sems[h]).wait()

    def gather_half(src_v, h):
        dst_v = out_bufs[h]

        @plsc.parallel_loop(0, HG // U)
        def _(gu):
            g0 = gu * U
            idxs = [perm_v[pl.ds((h * HG + g0 + u) * LANES, LANES)]
                    for u in range(U)]
            vals = [[plsc.load_gather(src_v, [row_ids[r], idxs[u]])
                     for r in range(R)] for u in range(U)]
            for u in range(U):
                for r in range(R):
                    dst_v[r, pl.ds((g0 + u) * LANES, LANES)] = vals[u][r]

    start_in(0, 0)
    start_in(1, 1)
    pltpu.sync_copy(perm_hbm, perm_v)
    wait_in(0, 0)
    for h in range(2):
        gather_half(in_bufs[0], h)
        start_out(0, h)
    start_in(2, 0)

    @pl.loop(0, (NCHUNK - 2) // 2)
    def _(cc):
        for b in range(2):
            c = 1 + cc * 2 + b
            bb = (1 + b) % 2
            wait_in(c, bb)
            for h in range(2):
                wait_out(c - 1, h)
                gather_half(in_bufs[bb], h)
                start_out(c, h)

            @pl.when(c + 2 < NCHUNK)
            def _():
                start_in(c + 2, bb)

    c = NCHUNK - 1
    wait_in(c, 1)
    for h in range(2):
        wait_out(c - 1, h)
        gather_half(in_bufs[1], h)
        start_out(c, h)
    for h in range(2):
        wait_out(c, h)


def kernel(x, permutation):
    return _permute(x, permutation)
